# baseline (speedup 1.0000x reference)
import functools

import jax
import jax.numpy as jnp
from jax import lax
from jax.experimental import pallas as pl
from jax.experimental.pallas import tpu as pltpu
from jax.experimental.pallas import tpu_sc as plsc

N = 10000
E = 320000
D = 128

NC = 2
NS = 16
NW = NC * NS
NPAD = 10240
RPS = NPAD // NS
EPW = E // NW

KC = 80
NCHUNK = EPW // KC
PASSES = 5
HALF = NCHUNK // PASSES

_MESH = plsc.VectorSubcoreMesh(core_axis_name="c", subcore_axis_name="s")


@functools.partial(
    pl.kernel,
    out_type=jax.ShapeDtypeStruct((NC, NS, RPS), jnp.float32),
    mesh=_MESH,
    scratch_types=[
        pltpu.VMEM((HALF, KC), jnp.int32),
        pltpu.VMEM((128,), jnp.float32),
        pltpu.VMEM((RPS,), jnp.float32),
        pltpu.VMEM_SHARED((NPAD,), jnp.float32),
        pltpu.SemaphoreType.DMA,
    ],
)
def _hist_kernel(dst_hbm, deg_out, idx_v, ones_v, zer_v, acc_sh, hsem):
    cid = lax.axis_index("c")
    sid = lax.axis_index("s")
    wid = sid * NC + cid

    for i in range(8):
        ones_v[pl.ds(i * 16, 16)] = jnp.ones((16,), jnp.float32)

    @pl.loop(0, RPS // 16)
    def _zero(i):
        zer_v[pl.ds(pl.multiple_of(i * 16, 16), 16)] = jnp.zeros(
            (16,), jnp.float32)

    pltpu.sync_copy(zer_v, acc_sh.at[pl.ds(sid * RPS, RPS)])
    plsc.subcore_barrier()

    for p in range(PASSES):
        pltpu.sync_copy(dst_hbm.at[wid, p], idx_v)

        @pl.loop(0, HALF // 5)
        def _accum(w):
            j = w * 5
            for k in range(5):
                pltpu.async_copy(ones_v.at[pl.ds(0, KC)],
                                 acc_sh.at[idx_v.at[j + k]], hsem, add=True)
            for k in range(5):
                pltpu.make_async_copy(ones_v.at[pl.ds(0, KC)],
                                      acc_sh.at[idx_v.at[j + k]],
                                      hsem).wait()

    plsc.subcore_barrier()
    pltpu.sync_copy(acc_sh.at[pl.ds(sid * RPS, RPS)], deg_out.at[cid, sid])


@functools.partial(
    pl.kernel,
    out_type=jax.ShapeDtypeStruct((NC, NS, RPS, D), jnp.float32),
    mesh=_MESH,
    scratch_types=[
        pltpu.VMEM((2, HALF, KC), jnp.int32),
        pltpu.VMEM((2, HALF, KC), jnp.int32),
        pltpu.VMEM((KC, D), jnp.float32),
        pltpu.VMEM((KC, D), jnp.float32),
        pltpu.VMEM((KC, D), jnp.float32),
        pltpu.VMEM_SHARED((NPAD, D), jnp.float32),
        pltpu.SemaphoreType.DMA,
        pltpu.SemaphoreType.DMA,
        pltpu.SemaphoreType.DMA,
        pltpu.SemaphoreType.DMA,
        pltpu.SemaphoreType.DMA,
        pltpu.SemaphoreType.DMA,
        pltpu.SemaphoreType.DMA,
    ],
)
def _edge_kernel(g_hbm, src_hbm, dst_hbm, acc_out,
                 src_v, dst_v, rows0, rows1, rows2, acc_sh,
                 gs0, gs1, gs2, ss0, ss1, ss2, isem):
    cid = lax.axis_index("c")
    sid = lax.axis_index("s")
    wid = sid * NC + cid

    pltpu.sync_copy(g_hbm.at[pl.ds(sid * RPS, RPS)],
                    acc_sh.at[pl.ds(sid * RPS, RPS)])
    pltpu.sync_copy(src_hbm.at[wid, 0], src_v.at[0])
    pltpu.sync_copy(dst_hbm.at[wid, 0], dst_v.at[0])
    plsc.subcore_barrier()

    for p in range(PASSES):
        s = p % 2
        sv = src_v.at[s]
        dv = dst_v.at[s]
        rows = (rows0, rows1, rows2)
        gsem = (gs0, gs1, gs2)
        ssem = (ss0, ss1, ss2)

        def start_g(j, b):
            pltpu.async_copy(g_hbm.at[sv.at[j]], rows[b], gsem[b])

        def wait_g(j, b):
            pltpu.make_async_copy(g_hbm.at[sv.at[j]], rows[b], gsem[b]).wait()

        def start_s(j, b):
            pltpu.async_copy(rows[b], acc_sh.at[dv.at[j]], ssem[b], add=True)

        def wait_s(j, b):
            pltpu.make_async_copy(rows[b], acc_sh.at[dv.at[j]],
                                  ssem[b]).wait()

        start_g(0, 0)
        start_g(1, 1)
        wait_g(0, 0)
        start_s(0, 0)
        start_g(2, 2)
        wait_g(1, 1)
        start_s(1, 1)

        if p + 1 < PASSES:
            pltpu.async_copy(src_hbm.at[wid, p + 1], src_v.at[1 - s], isem)
            pltpu.async_copy(dst_hbm.at[wid, p + 1], dst_v.at[1 - s], isem)

        wait_s(0, 0)
        start_g(3, 0)
        wait_g(2, 2)
        start_s(2, 2)

        @pl.loop(0, (HALF - 4) // 3)
        def _pipe(i):
            j = 4 + i * 3
            wait_s(j - 3, 1)
            start_g(j, 1)
            wait_g(j - 1, 0)
            start_s(j - 1, 0)

            wait_s(j - 2, 2)
            start_g(j + 1, 2)
            wait_g(j, 1)
            start_s(j, 1)

            wait_s(j - 1, 0)
            start_g(j + 2, 0)
            wait_g(j + 1, 2)
            start_s(j + 1, 2)

        wait_g(HALF - 1, 0)
        start_s(HALF - 1, 0)
        wait_s(HALF - 3, 1)
        wait_s(HALF - 2, 2)
        wait_s(HALF - 1, 0)

        if p + 1 < PASSES:
            pltpu.make_async_copy(
                src_hbm.at[wid, p + 1], src_v.at[1 - s], isem).wait()
            pltpu.make_async_copy(
                dst_hbm.at[wid, p + 1], dst_v.at[1 - s], isem).wait()

    plsc.subcore_barrier()
    pltpu.sync_copy(acc_sh.at[pl.ds(sid * RPS, RPS)], acc_out.at[cid, sid])


def _g_body(x_ref, w_ref, deg_ref, g_ref):
    h = jnp.dot(x_ref[...], w_ref[...], preferred_element_type=jnp.float32)
    d = deg_ref[...]
    dinv = lax.rsqrt(d[0] + d[1] + 1.0)
    g_ref[...] = h * dinv[:, None]


def _final_body(acc_ref, g_ref, deg_ref, b_ref, a_ref, out_ref):
    d = deg_ref[...]
    dinv = lax.rsqrt(d[0] + d[1] + 1.0)
    a = acc_ref[...]
    s = (a[0] + a[1] - g_ref[...]) * dinv[:, None] + b_ref[...]
    out_ref[...] = jnp.where(s > 0, s, a_ref[...] * s)


_BLK = 1024
_GRID = NPAD // _BLK


def kernel(x, edge_index, W, b, prelu_a):
    src = edge_index[0].reshape(NW, PASSES, HALF, KC)
    dst = edge_index[1].reshape(NW, PASSES, HALF, KC)

    deg = _hist_kernel(dst)
    deg2 = deg.reshape(NC, NPAD)

    g = pl.pallas_call(
        _g_body,
        grid=(_GRID,),
        in_specs=[
            pl.BlockSpec((_BLK, D), lambda i: (i, 0)),
            pl.BlockSpec((D, D), lambda i: (0, 0)),
            pl.BlockSpec((NC, _BLK), lambda i: (0, i)),
        ],
        out_specs=pl.BlockSpec((_BLK, D), lambda i: (i, 0)),
        out_shape=jax.ShapeDtypeStruct((NPAD, D), jnp.float32),
    )(x, W, deg2)

    acc = _edge_kernel(g, src, dst)
    acc = acc.reshape(NC, NPAD, D)

    out = pl.pallas_call(
        _final_body,
        grid=(_GRID,),
        in_specs=[
            pl.BlockSpec((NC, _BLK, D), lambda i: (0, i, 0)),
            pl.BlockSpec((_BLK, D), lambda i: (i, 0)),
            pl.BlockSpec((NC, _BLK), lambda i: (0, i)),
            pl.BlockSpec((1, D), lambda i: (0, 0)),
            pl.BlockSpec((1, D), lambda i: (0, 0)),
        ],
        out_specs=pl.BlockSpec((_BLK, D), lambda i: (i, 0)),
        out_shape=jax.ShapeDtypeStruct((N, D), jnp.float32),
    )(acc, g, deg2, b.reshape(1, D), prelu_a.reshape(1, D))
    return out


# speedup vs baseline: 1.0253x; 1.0253x over previous
import functools

import jax
import jax.numpy as jnp
from jax import lax
from jax.experimental import pallas as pl
from jax.experimental.pallas import tpu as pltpu
from jax.experimental.pallas import tpu_sc as plsc

N = 10000
E = 320000
D = 128

NC = 2
NS = 16
NW = NC * NS
NPAD = 10240
RPS = NPAD // NS
EPW = E // NW

KC = 80
NCHUNK = EPW // KC
PASSES = 5
HALF = NCHUNK // PASSES

_MESH = plsc.VectorSubcoreMesh(core_axis_name="c", subcore_axis_name="s")


@functools.partial(
    pl.kernel,
    out_type=jax.ShapeDtypeStruct((NC, NS, RPS), jnp.float32),
    mesh=_MESH,
    scratch_types=[
        pltpu.VMEM((HALF, KC), jnp.int32),
        pltpu.VMEM((128,), jnp.float32),
        pltpu.VMEM((RPS,), jnp.float32),
        pltpu.VMEM_SHARED((NPAD,), jnp.float32),
        pltpu.SemaphoreType.DMA,
    ],
)
def _hist_kernel(dst_hbm, deg_out, idx_v, ones_v, zer_v, acc_sh, hsem):
    cid = lax.axis_index("c")
    sid = lax.axis_index("s")
    wid = sid * NC + cid

    for i in range(8):
        ones_v[pl.ds(i * 16, 16)] = jnp.ones((16,), jnp.float32)

    @pl.loop(0, RPS // 16)
    def _zero(i):
        zer_v[pl.ds(pl.multiple_of(i * 16, 16), 16)] = jnp.zeros(
            (16,), jnp.float32)

    pltpu.sync_copy(zer_v, acc_sh.at[pl.ds(sid * RPS, RPS)])
    plsc.subcore_barrier()

    for p in range(PASSES):
        pltpu.sync_copy(dst_hbm.at[wid, p], idx_v)

        @pl.loop(0, HALF // 5)
        def _accum(w):
            j = w * 5
            for k in range(5):
                pltpu.async_copy(ones_v.at[pl.ds(0, KC)],
                                 acc_sh.at[idx_v.at[j + k]], hsem, add=True)
            for k in range(5):
                pltpu.make_async_copy(ones_v.at[pl.ds(0, KC)],
                                      acc_sh.at[idx_v.at[j + k]],
                                      hsem).wait()

    plsc.subcore_barrier()
    pltpu.sync_copy(acc_sh.at[pl.ds(sid * RPS, RPS)], deg_out.at[cid, sid])


@functools.partial(
    pl.kernel,
    out_type=jax.ShapeDtypeStruct((NC, NS, RPS, D), jnp.float32),
    mesh=_MESH,
    scratch_types=[
        pltpu.VMEM((2, HALF, KC), jnp.int32),
        pltpu.VMEM((2, HALF, KC), jnp.int32),
        pltpu.VMEM((KC, D), jnp.float32),
        pltpu.VMEM((KC, D), jnp.float32),
        pltpu.VMEM((KC, D), jnp.float32),
        pltpu.VMEM_SHARED((NPAD, D), jnp.float32),
        pltpu.SemaphoreType.DMA,
        pltpu.SemaphoreType.DMA,
        pltpu.SemaphoreType.DMA,
        pltpu.SemaphoreType.DMA,
        pltpu.SemaphoreType.DMA,
        pltpu.SemaphoreType.DMA,
        pltpu.SemaphoreType.DMA,
    ],
)
def _edge_kernel(g_hbm, src_hbm, dst_hbm, acc_out,
                 src_v, dst_v, rows0, rows1, rows2, acc_sh,
                 gs0, gs1, gs2, ss0, ss1, ss2, isem):
    cid = lax.axis_index("c")
    sid = lax.axis_index("s")
    wid = sid * NC + cid

    pltpu.sync_copy(g_hbm.at[pl.ds(sid * RPS, RPS)],
                    acc_sh.at[pl.ds(sid * RPS, RPS)])
    pltpu.sync_copy(src_hbm.at[wid, 0], src_v.at[0])
    pltpu.sync_copy(dst_hbm.at[wid, 0], dst_v.at[0])
    plsc.subcore_barrier()

    for p in range(PASSES):
        s = p % 2
        sv = src_v.at[s]
        dv = dst_v.at[s]
        rows = (rows0, rows1, rows2)
        gsem = (gs0, gs1, gs2)
        ssem = (ss0, ss1, ss2)

        def start_g(j, b):
            pltpu.async_copy(g_hbm.at[sv.at[j]], rows[b], gsem[b])

        def wait_g(j, b):
            pltpu.make_async_copy(g_hbm.at[sv.at[j]], rows[b], gsem[b]).wait()

        def start_s(j, b):
            pltpu.async_copy(rows[b], acc_sh.at[dv.at[j]], ssem[b], add=True)

        def wait_s(j, b):
            pltpu.make_async_copy(rows[b], acc_sh.at[dv.at[j]],
                                  ssem[b]).wait()

        start_g(0, 0)
        start_g(1, 1)
        wait_g(0, 0)
        start_s(0, 0)
        start_g(2, 2)
        wait_g(1, 1)
        start_s(1, 1)

        if p + 1 < PASSES:
            pltpu.async_copy(src_hbm.at[wid, p + 1], src_v.at[1 - s], isem)
            pltpu.async_copy(dst_hbm.at[wid, p + 1], dst_v.at[1 - s], isem)

        wait_s(0, 0)
        start_g(3, 0)
        wait_g(2, 2)
        start_s(2, 2)

        @pl.loop(0, (HALF - 4) // 3)
        def _pipe(i):
            j = 4 + i * 3
            wait_s(j - 3, 1)
            start_g(j, 1)
            wait_g(j - 1, 0)
            start_s(j - 1, 0)

            wait_s(j - 2, 2)
            start_g(j + 1, 2)
            wait_g(j, 1)
            start_s(j, 1)

            wait_s(j - 1, 0)
            start_g(j + 2, 0)
            wait_g(j + 1, 2)
            start_s(j + 1, 2)

        wait_g(HALF - 1, 0)
        start_s(HALF - 1, 0)
        wait_s(HALF - 3, 1)
        wait_s(HALF - 2, 2)
        wait_s(HALF - 1, 0)

        if p + 1 < PASSES:
            pltpu.make_async_copy(
                src_hbm.at[wid, p + 1], src_v.at[1 - s], isem).wait()
            pltpu.make_async_copy(
                dst_hbm.at[wid, p + 1], dst_v.at[1 - s], isem).wait()

    plsc.subcore_barrier()
    pltpu.sync_copy(acc_sh.at[pl.ds(sid * RPS, RPS)], acc_out.at[cid, sid])


def _g_body(x_ref, w_ref, deg_ref, g_ref):
    h = jnp.dot(x_ref[...], w_ref[...], preferred_element_type=jnp.float32)
    d = deg_ref[...]
    dinv = lax.rsqrt(d[0] + d[1] + 1.0)
    g_ref[...] = h * dinv[:, None]


def _final_body(acc_ref, g_ref, deg_ref, b_ref, a_ref, out_ref):
    d = deg_ref[...]
    dinv = lax.rsqrt(d[0] + d[1] + 1.0)
    a = acc_ref[...]
    s = (a[0] + a[1] - g_ref[...]) * dinv[:, None] + b_ref[...]
    out_ref[...] = jnp.where(s > 0, s, a_ref[...] * s)


_BLK = 2048
_GRID = NPAD // _BLK


def kernel(x, edge_index, W, b, prelu_a):
    src = edge_index[0].reshape(NW, PASSES, HALF, KC)
    dst = edge_index[1].reshape(NW, PASSES, HALF, KC)

    deg = _hist_kernel(dst)
    deg2 = deg.reshape(NC, NPAD)

    g = pl.pallas_call(
        _g_body,
        grid=(_GRID,),
        in_specs=[
            pl.BlockSpec((_BLK, D), lambda i: (i, 0)),
            pl.BlockSpec((D, D), lambda i: (0, 0)),
            pl.BlockSpec((NC, _BLK), lambda i: (0, i)),
        ],
        out_specs=pl.BlockSpec((_BLK, D), lambda i: (i, 0)),
        out_shape=jax.ShapeDtypeStruct((NPAD, D), jnp.float32),
    )(x, W, deg2)

    acc = _edge_kernel(g, src, dst)
    acc = acc.reshape(NC, NPAD, D)

    out = pl.pallas_call(
        _final_body,
        grid=(_GRID,),
        in_specs=[
            pl.BlockSpec((NC, _BLK, D), lambda i: (0, i, 0)),
            pl.BlockSpec((_BLK, D), lambda i: (i, 0)),
            pl.BlockSpec((NC, _BLK), lambda i: (0, i)),
            pl.BlockSpec((1, D), lambda i: (0, 0)),
            pl.BlockSpec((1, D), lambda i: (0, 0)),
        ],
        out_specs=pl.BlockSpec((_BLK, D), lambda i: (i, 0)),
        out_shape=jax.ShapeDtypeStruct((N, D), jnp.float32),
    )(acc, g, deg2, b.reshape(1, D), prelu_a.reshape(1, D))
    return out
